# baseline (speedup 1.0000x reference)
import functools

import jax
import jax.numpy as jnp
from jax import lax
from jax.experimental import pallas as pl
from jax.experimental.pallas import tpu as pltpu
from jax.experimental.pallas import tpu_sc as plsc

_NW = 32
_LANES = 16
_COLS_PER_CHUNK = 128


@functools.partial(jax.jit, static_argnums=(0, 1))
def _lut_gather(n_rows, n_cols, codes, lut32):
    cols_per_w = n_cols // _NW
    n_chunks = cols_per_w // _COLS_PER_CHUNK
    n_j = _COLS_PER_CHUNK // _LANES
    mesh = plsc.VectorSubcoreMesh(core_axis_name="c", subcore_axis_name="s")

    @functools.partial(
        pl.kernel,
        out_type=jax.ShapeDtypeStruct((n_rows, n_cols), jnp.int32),
        mesh=mesh,
        compiler_params=pltpu.CompilerParams(
            needs_layout_passes=False, use_tc_tiling_on_sc=True),
        scratch_types=[
            pltpu.VMEM((256,), jnp.int32),
            pltpu.VMEM((n_rows, _COLS_PER_CHUNK), jnp.int32),
            pltpu.VMEM((n_rows, _COLS_PER_CHUNK), jnp.int32),
            pltpu.VMEM((n_rows, _COLS_PER_CHUNK), jnp.int32),
            pltpu.VMEM((n_rows, _COLS_PER_CHUNK), jnp.int32),
            pltpu.SemaphoreType.DMA,
            pltpu.SemaphoreType.DMA,
            pltpu.SemaphoreType.DMA,
            pltpu.SemaphoreType.DMA,
        ],
    )
    def k(codes_hbm, lut_hbm, out_hbm, lut_v, in_v0, in_v1, out_v0, out_v1,
          isem0, isem1, osem0, osem1):
        wid = lax.axis_index("s") * 2 + lax.axis_index("c")
        base_col = wid * cols_per_w
        pltpu.sync_copy(lut_hbm, lut_v)
        in_bufs = (in_v0, in_v1)
        out_bufs = (out_v0, out_v1)
        isems = (isem0, isem1)
        osems = (osem0, osem1)
        in_cps = [None, None]
        out_cps = [None, None]

        def start_in(g):
            b = g % 2
            in_cps[b] = pltpu.async_copy(
                codes_hbm.at[:, pl.ds(base_col + g * _COLS_PER_CHUNK,
                                      _COLS_PER_CHUNK)],
                in_bufs[b], isems[b])

        start_in(0)
        for g in range(n_chunks):
            b = g % 2
            if g + 1 < n_chunks:
                start_in(g + 1)
            in_cps[b].wait()
            if out_cps[b] is not None:
                out_cps[b].wait()
            in_v, out_v = in_bufs[b], out_bufs[b]

            @plsc.parallel_loop(0, n_rows, 1, unroll=2)
            def body(p):
                for j in range(n_j):
                    idx = in_v[p, pl.ds(j * _LANES, _LANES)]
                    out_v[p, pl.ds(j * _LANES, _LANES)] = plsc.load_gather(
                        lut_v, [idx])

            out_cps[b] = pltpu.async_copy(
                out_bufs[b],
                out_hbm.at[:, pl.ds(base_col + g * _COLS_PER_CHUNK,
                                    _COLS_PER_CHUNK)], osems[b])

        for b in range(2):
            if out_cps[b] is not None:
                out_cps[b].wait()

    return k(codes, lut32)


def kernel(char_bytes, lut):
    B, L = char_bytes.shape
    lut32 = lut.astype(jnp.int32)
    codes_t = char_bytes.astype(jnp.int32).T
    out_t = _lut_gather(L, B, codes_t, lut32)
    return out_t.T.astype(lut.dtype)
